# baseline (speedup 1.0000x reference)
import jax
import jax.numpy as jnp
from jax import lax
from jax.experimental import pallas as pl
from jax.experimental.pallas import tpu as pltpu

_VOCAB = 1000000
_DIM = 64
_CTX = 200
_GPC = 50
_GSTEPS = _CTX // _GPC
_LASTBLK = _VOCAB // 128 - 1
_TAIL0 = (_VOCAB // 128) * 128
_NQ = 8
_CB = 10240
_MSTEPS = 12
_QSPAN = _MSTEPS * _CB
_MAIN = _NQ * _QSPAN
_TAILN = _VOCAB - _MAIN
_TCB = 1536
_TBLKS = (_TAIL0 - _MAIN) // _TCB
_TOFF = _MAIN // _TCB


def _gather_body(idx_ref, *refs):
    e_refs = refs[:_GPC]
    et_ref, xv_ref, xacc = refs[_GPC:]
    i = pl.program_id(0)

    @pl.when(i == 0)
    def _():
        xacc[...] = jnp.zeros((_DIM, 1), jnp.float32)

    lane = lax.broadcasted_iota(jnp.int32, (1, 128), 1)
    lane64 = lax.broadcasted_iota(jnp.int32, (1, 64), 1)
    total = xacc[...]
    for q in range(_GPC):
        v = idx_ref[i * _GPC + q]
        sel = jnp.where(lane == v % 128, e_refs[q][...], 0.0)
        col = jnp.sum(sel, axis=1, keepdims=True)
        selt = jnp.where(lane64 == v - _TAIL0, et_ref[...], 0.0)
        colt = jnp.sum(selt, axis=1, keepdims=True)
        total = total + jnp.where(v >= _TAIL0, colt, col)
    xacc[...] = total

    @pl.when(i == _GSTEPS - 1)
    def _():
        xv_ref[...] = total


def _decode_body(xv_ref, *refs):
    w_refs = refs[:_NQ]
    b_refs = refs[_NQ:2 * _NQ]
    out_refs = refs[2 * _NQ:3 * _NQ]
    m_out, s_out, m_ref, s_ref = refs[3 * _NQ:]
    i = pl.program_id(0)

    @pl.when(i == 0)
    def _():
        m_ref[...] = jnp.full((1, 1), -jnp.inf, jnp.float32)
        s_ref[...] = jnp.zeros((1, 1), jnp.float32)

    x = xv_ref[...]
    accs = []
    for q in range(_NQ):
        acc = (jnp.sum(w_refs[q][...] * x, axis=0, keepdims=True)
               + b_refs[q][...].reshape(1, _CB))
        out_refs[q][...] = acc
        accs.append(acc)
    allacc = jnp.concatenate(accs, axis=1)

    m_old = m_ref[...]
    bmax = jnp.max(allacc, axis=(0, 1), keepdims=True)
    m_new = jnp.maximum(m_old, bmax)
    s_new = (s_ref[...] * jnp.exp(m_old - m_new)
             + jnp.sum(jnp.exp(allacc - m_new), axis=(0, 1), keepdims=True))
    s_ref[...] = s_new
    m_ref[...] = m_new

    @pl.when(i == _MSTEPS - 1)
    def _():
        m_out[...] = m_new
        s_out[...] = s_new


def _tail_body(xv_ref, *refs):
    w_refs = refs[:_TBLKS]
    w64_ref, bt_ref, m_ref, s_ref, lp_ref, lse_ref = refs[_TBLKS:]
    x = xv_ref[...]
    parts = [jnp.sum(w_refs[j][...] * x, axis=0, keepdims=True)
             for j in range(_TBLKS)]
    parts.append(jnp.sum(w64_ref[...] * x, axis=0, keepdims=True))
    acc = jnp.concatenate(parts, axis=1) + bt_ref[...]
    m_old = m_ref[...]
    m_new = jnp.maximum(m_old, jnp.max(acc, axis=(0, 1), keepdims=True))
    s_new = (s_ref[...] * jnp.exp(m_old - m_new)
             + jnp.sum(jnp.exp(acc - m_new), axis=(0, 1), keepdims=True))
    lse = m_new + jnp.log(s_new)
    lse_ref[...] = lse
    lp_ref[...] = acc - lse


def _sub_body(*refs):
    in_refs = refs[:_NQ]
    lse_ref = refs[_NQ]
    out_refs = refs[_NQ + 1:]
    for q in range(_NQ):
        out_refs[q][...] = in_refs[q][...] - lse_ref[...]


_SUBBLK = _QSPAN // 3


def kernel(inputs, encode_weight, decode_weight, decode_bias):
    idx = inputs.astype(jnp.int32)
    enc_t = encode_weight.T
    dec_t = decode_weight.T
    enc_tail = enc_t[:, _TAIL0:]
    dec_tail64 = dec_t[:, _TAIL0:]
    b_tail = decode_bias[_MAIN:].reshape(1, _TAILN)

    def e_spec(q):
        return pl.BlockSpec(
            (_DIM, 128),
            lambda i, idxp, q=q: (
                0, jnp.minimum(idxp[i * _GPC + q] // 128, _LASTBLK)))

    xv = pl.pallas_call(
        _gather_body,
        grid_spec=pltpu.PrefetchScalarGridSpec(
            num_scalar_prefetch=1,
            grid=(_GSTEPS,),
            in_specs=[e_spec(q) for q in range(_GPC)] + [
                pl.BlockSpec((_DIM, 64), lambda i, idxp: (0, 0)),
            ],
            out_specs=pl.BlockSpec((_DIM, 1), lambda i, idxp: (0, 0)),
            scratch_shapes=[pltpu.VMEM((_DIM, 1), jnp.float32)],
        ),
        out_shape=jax.ShapeDtypeStruct((_DIM, 1), jnp.float32),
        compiler_params=pltpu.CompilerParams(
            dimension_semantics=("arbitrary",),
        ),
    )(idx, *([enc_t] * _GPC), enc_tail)

    def w_spec(q):
        return pl.BlockSpec(
            (_DIM, _CB), lambda i, q=q: (0, q * _MSTEPS + i))

    def b_spec(q):
        return pl.BlockSpec((_CB,), lambda i, q=q: (q * _MSTEPS + i,))

    outs = pl.pallas_call(
        _decode_body,
        grid=(_MSTEPS,),
        in_specs=[
            pl.BlockSpec((_DIM, 1), lambda i: (0, 0)),
        ] + [w_spec(q) for q in range(_NQ)]
          + [b_spec(q) for q in range(_NQ)],
        out_specs=[
            pl.BlockSpec((1, _CB), lambda i: (0, i)) for _ in range(_NQ)
        ] + [
            pl.BlockSpec((1, 1), lambda i: (0, 0)),
            pl.BlockSpec((1, 1), lambda i: (0, 0)),
        ],
        out_shape=[
            jax.ShapeDtypeStruct((1, _QSPAN), jnp.float32)
            for _ in range(_NQ)
        ] + [
            jax.ShapeDtypeStruct((1, 1), jnp.float32),
            jax.ShapeDtypeStruct((1, 1), jnp.float32),
        ],
        scratch_shapes=[
            pltpu.VMEM((1, 1), jnp.float32),
            pltpu.VMEM((1, 1), jnp.float32),
        ],
        compiler_params=pltpu.CompilerParams(
            dimension_semantics=("arbitrary",),
        ),
    )(xv, *([dec_t] * _NQ), *([decode_bias] * _NQ))
    logit_qs, m_run, s_run = outs[:_NQ], outs[_NQ], outs[_NQ + 1]

    def wt_spec(j):
        return pl.BlockSpec((_DIM, _TCB), lambda i, j=j: (0, _TOFF + j))

    lp_tail, lse = pl.pallas_call(
        _tail_body,
        grid=(1,),
        in_specs=[
            pl.BlockSpec((_DIM, 1), lambda i: (0, 0)),
        ] + [wt_spec(j) for j in range(_TBLKS)] + [
            pl.BlockSpec((_DIM, 64), lambda i: (0, 0)),
            pl.BlockSpec((1, _TAILN), lambda i: (0, 0)),
            pl.BlockSpec((1, 1), lambda i: (0, 0)),
            pl.BlockSpec((1, 1), lambda i: (0, 0)),
        ],
        out_specs=[
            pl.BlockSpec((1, _TAILN), lambda i: (0, 0)),
            pl.BlockSpec((1, 1), lambda i: (0, 0)),
        ],
        out_shape=[
            jax.ShapeDtypeStruct((1, _TAILN), jnp.float32),
            jax.ShapeDtypeStruct((1, 1), jnp.float32),
        ],
    )(xv, *([dec_t] * _TBLKS), dec_tail64, b_tail, m_run, s_run)

    lp_qs = pl.pallas_call(
        _sub_body,
        grid=(3,),
        in_specs=[
            pl.BlockSpec((1, _SUBBLK), lambda i: (0, i))
            for _ in range(_NQ)
        ] + [pl.BlockSpec((1, 1), lambda i: (0, 0))],
        out_specs=[
            pl.BlockSpec((1, _SUBBLK), lambda i: (0, i))
            for _ in range(_NQ)
        ],
        out_shape=[
            jax.ShapeDtypeStruct((1, _QSPAN), jnp.float32)
            for _ in range(_NQ)
        ],
    )(*logit_qs, lse)

    return jnp.concatenate(list(lp_qs) + [lp_tail], axis=1)


# speedup vs baseline: 1.0051x; 1.0051x over previous
import jax
import jax.numpy as jnp
from jax import lax
from jax.experimental import pallas as pl
from jax.experimental.pallas import tpu as pltpu

_VOCAB = 1000000
_DIM = 64
_CTX = 200
_GPC = 50
_GSTEPS = _CTX // _GPC
_LASTBLK = _VOCAB // 128 - 1
_TAIL0 = (_VOCAB // 128) * 128
_NQ = 4
_CB = 20480
_MSTEPS = 12
_QSPAN = _MSTEPS * _CB
_MAIN = _NQ * _QSPAN
_TAILN = _VOCAB - _MAIN
_TCB = 1536
_TBLKS = (_TAIL0 - _MAIN) // _TCB
_TOFF = _MAIN // _TCB


def _gather_body(idx_ref, *refs):
    e_refs = refs[:_GPC]
    et_ref, xv_ref, xacc = refs[_GPC:]
    i = pl.program_id(0)

    @pl.when(i == 0)
    def _():
        xacc[...] = jnp.zeros((_DIM, 1), jnp.float32)

    lane = lax.broadcasted_iota(jnp.int32, (1, 128), 1)
    lane64 = lax.broadcasted_iota(jnp.int32, (1, 64), 1)
    total = xacc[...]
    for q in range(_GPC):
        v = idx_ref[i * _GPC + q]
        sel = jnp.where(lane == v % 128, e_refs[q][...], 0.0)
        col = jnp.sum(sel, axis=1, keepdims=True)
        selt = jnp.where(lane64 == v - _TAIL0, et_ref[...], 0.0)
        colt = jnp.sum(selt, axis=1, keepdims=True)
        total = total + jnp.where(v >= _TAIL0, colt, col)
    xacc[...] = total

    @pl.when(i == _GSTEPS - 1)
    def _():
        xv_ref[...] = total


def _decode_body(xv_ref, *refs):
    w_refs = refs[:_NQ]
    b_refs = refs[_NQ:2 * _NQ]
    out_refs = refs[2 * _NQ:3 * _NQ]
    m_out, s_out, m_ref, s_ref = refs[3 * _NQ:]
    i = pl.program_id(0)

    @pl.when(i == 0)
    def _():
        m_ref[...] = jnp.full((1, 1), -jnp.inf, jnp.float32)
        s_ref[...] = jnp.zeros((1, 1), jnp.float32)

    x = xv_ref[...]
    accs = []
    for q in range(_NQ):
        acc = (jnp.sum(w_refs[q][...] * x, axis=0, keepdims=True)
               + b_refs[q][...].reshape(1, _CB))
        out_refs[q][...] = acc
        accs.append(acc)
    allacc = jnp.concatenate(accs, axis=1)

    m_old = m_ref[...]
    bmax = jnp.max(allacc, axis=(0, 1), keepdims=True)
    m_new = jnp.maximum(m_old, bmax)
    s_new = (s_ref[...] * jnp.exp(m_old - m_new)
             + jnp.sum(jnp.exp(allacc - m_new), axis=(0, 1), keepdims=True))
    s_ref[...] = s_new
    m_ref[...] = m_new

    @pl.when(i == _MSTEPS - 1)
    def _():
        m_out[...] = m_new
        s_out[...] = s_new


def _tail_body(xv_ref, *refs):
    w_refs = refs[:_TBLKS]
    w64_ref, bt_ref, m_ref, s_ref, lp_ref, lse_ref = refs[_TBLKS:]
    x = xv_ref[...]
    parts = [jnp.sum(w_refs[j][...] * x, axis=0, keepdims=True)
             for j in range(_TBLKS)]
    parts.append(jnp.sum(w64_ref[...] * x, axis=0, keepdims=True))
    acc = jnp.concatenate(parts, axis=1) + bt_ref[...]
    m_old = m_ref[...]
    m_new = jnp.maximum(m_old, jnp.max(acc, axis=(0, 1), keepdims=True))
    s_new = (s_ref[...] * jnp.exp(m_old - m_new)
             + jnp.sum(jnp.exp(acc - m_new), axis=(0, 1), keepdims=True))
    lse = m_new + jnp.log(s_new)
    lse_ref[...] = lse
    lp_ref[...] = acc - lse


def _sub_body(*refs):
    in_refs = refs[:_NQ]
    lse_ref = refs[_NQ]
    out_refs = refs[_NQ + 1:]
    for q in range(_NQ):
        out_refs[q][...] = in_refs[q][...] - lse_ref[...]


_SUBBLK = _QSPAN // 3


def kernel(inputs, encode_weight, decode_weight, decode_bias):
    idx = inputs.astype(jnp.int32)
    enc_t = encode_weight.T
    dec_t = decode_weight.T
    enc_tail = enc_t[:, _TAIL0:]
    dec_tail64 = dec_t[:, _TAIL0:]
    b_tail = decode_bias[_MAIN:].reshape(1, _TAILN)

    def e_spec(q):
        return pl.BlockSpec(
            (_DIM, 128),
            lambda i, idxp, q=q: (
                0, jnp.minimum(idxp[i * _GPC + q] // 128, _LASTBLK)))

    xv = pl.pallas_call(
        _gather_body,
        grid_spec=pltpu.PrefetchScalarGridSpec(
            num_scalar_prefetch=1,
            grid=(_GSTEPS,),
            in_specs=[e_spec(q) for q in range(_GPC)] + [
                pl.BlockSpec((_DIM, 64), lambda i, idxp: (0, 0)),
            ],
            out_specs=pl.BlockSpec((_DIM, 1), lambda i, idxp: (0, 0)),
            scratch_shapes=[pltpu.VMEM((_DIM, 1), jnp.float32)],
        ),
        out_shape=jax.ShapeDtypeStruct((_DIM, 1), jnp.float32),
        compiler_params=pltpu.CompilerParams(
            dimension_semantics=("arbitrary",),
        ),
    )(idx, *([enc_t] * _GPC), enc_tail)

    def w_spec(q):
        return pl.BlockSpec(
            (_DIM, _CB), lambda i, q=q: (0, q * _MSTEPS + i))

    def b_spec(q):
        return pl.BlockSpec((_CB,), lambda i, q=q: (q * _MSTEPS + i,))

    outs = pl.pallas_call(
        _decode_body,
        grid=(_MSTEPS,),
        in_specs=[
            pl.BlockSpec((_DIM, 1), lambda i: (0, 0)),
        ] + [w_spec(q) for q in range(_NQ)]
          + [b_spec(q) for q in range(_NQ)],
        out_specs=[
            pl.BlockSpec((1, _CB), lambda i: (0, i)) for _ in range(_NQ)
        ] + [
            pl.BlockSpec((1, 1), lambda i: (0, 0)),
            pl.BlockSpec((1, 1), lambda i: (0, 0)),
        ],
        out_shape=[
            jax.ShapeDtypeStruct((1, _QSPAN), jnp.float32)
            for _ in range(_NQ)
        ] + [
            jax.ShapeDtypeStruct((1, 1), jnp.float32),
            jax.ShapeDtypeStruct((1, 1), jnp.float32),
        ],
        scratch_shapes=[
            pltpu.VMEM((1, 1), jnp.float32),
            pltpu.VMEM((1, 1), jnp.float32),
        ],
        compiler_params=pltpu.CompilerParams(
            dimension_semantics=("arbitrary",),
        ),
    )(xv, *([dec_t] * _NQ), *([decode_bias] * _NQ))
    logit_qs, m_run, s_run = outs[:_NQ], outs[_NQ], outs[_NQ + 1]

    def wt_spec(j):
        return pl.BlockSpec((_DIM, _TCB), lambda i, j=j: (0, _TOFF + j))

    lp_tail, lse = pl.pallas_call(
        _tail_body,
        grid=(1,),
        in_specs=[
            pl.BlockSpec((_DIM, 1), lambda i: (0, 0)),
        ] + [wt_spec(j) for j in range(_TBLKS)] + [
            pl.BlockSpec((_DIM, 64), lambda i: (0, 0)),
            pl.BlockSpec((1, _TAILN), lambda i: (0, 0)),
            pl.BlockSpec((1, 1), lambda i: (0, 0)),
            pl.BlockSpec((1, 1), lambda i: (0, 0)),
        ],
        out_specs=[
            pl.BlockSpec((1, _TAILN), lambda i: (0, 0)),
            pl.BlockSpec((1, 1), lambda i: (0, 0)),
        ],
        out_shape=[
            jax.ShapeDtypeStruct((1, _TAILN), jnp.float32),
            jax.ShapeDtypeStruct((1, 1), jnp.float32),
        ],
    )(xv, *([dec_t] * _TBLKS), dec_tail64, b_tail, m_run, s_run)

    lp_qs = pl.pallas_call(
        _sub_body,
        grid=(3,),
        in_specs=[
            pl.BlockSpec((1, _SUBBLK), lambda i: (0, i))
            for _ in range(_NQ)
        ] + [pl.BlockSpec((1, 1), lambda i: (0, 0))],
        out_specs=[
            pl.BlockSpec((1, _SUBBLK), lambda i: (0, i))
            for _ in range(_NQ)
        ],
        out_shape=[
            jax.ShapeDtypeStruct((1, _QSPAN), jnp.float32)
            for _ in range(_NQ)
        ],
    )(*logit_qs, lse)

    return jnp.concatenate(list(lp_qs) + [lp_tail], axis=1)
